# baseline (speedup 1.0000x reference)
import functools

import jax
import jax.numpy as jnp
from jax import lax
from jax.experimental import pallas as pl
from jax.experimental.pallas import tpu as pltpu
from jax.experimental.pallas import tpu_sc as plsc

N = 10000
E = 320000
SEQ = 8
IN_DIM = 16
H = 128

NC = 2
NS = 16
NW = NC * NS
EPW = E // NW
NPAD = 10240
RPT = NPAD // NS
CH = 128
NFULL = EPW // CH
TAIL = EPW - NFULL * CH

R = 1000


def _sc_mesh():
    return plsc.VectorSubcoreMesh(
        core_axis_name="c", subcore_axis_name="s", num_cores=NC, num_subcores=NS
    )


def _deg_kernel(dst_hbm, w_hbm, zeros_hbm, out_hbm, dacc,
                dv0, dv1, dv2, dv3, wv0, wv1, wv2, wv3, zbuf,
                dl0, dl1, dl2, dl3, ds0, ds1, ds2, ds3):
    cid = lax.axis_index("c")
    sid = lax.axis_index("s")
    wid = cid * NS + sid
    e0 = wid * CPW * CH

    SETS = ((dv0, wv0, dl0, ds0), (dv1, wv1, dl1, ds1),
            (dv2, wv2, dl2, ds2), (dv3, wv3, dl3, ds3))

    def loads(b, st):
        dv, wv, sl, _ = st
        off = e0 + CH * b
        pltpu.async_copy(dst_hbm.at[pl.ds(off, CH)], dv, sl)
        pltpu.async_copy(w_hbm.at[pl.ds(off, CH)], wv, sl)

    def wait_loads(st):
        dv, wv, sl, _ = st
        pltpu.make_async_copy(dst_hbm.at[pl.ds(0, CH)], dv, sl).wait()
        pltpu.make_async_copy(w_hbm.at[pl.ds(0, CH)], wv, sl).wait()

    def scat(st):
        dv, wv, _, ss = st
        pltpu.async_copy(wv, dacc.at[dv], ss, add=True)

    def wait_scat(st):
        dv, wv, _, ss = st
        pltpu.make_async_copy(wv, dacc.at[dv], ss).wait()

    for k in range(4):
        loads(k, SETS[k])
    pltpu.sync_copy(zeros_hbm.at[pl.ds(0, RPT)], zbuf)
    pltpu.sync_copy(zbuf, dacc.at[pl.ds(sid * RPT, RPT)])
    plsc.subcore_barrier()

    def quad(q, _):
        b = 4 * q
        for k in range(4):
            wait_loads(SETS[k])
            scat(SETS[k])
        for k in range(4):
            wait_scat(SETS[k])
            loads(b + 4 + k, SETS[k])
        return _

    lax.fori_loop(0, CPW // 4 - 1, quad, None)
    for k in range(4):
        wait_loads(SETS[k])
        scat(SETS[k])
    for k in range(4):
        wait_scat(SETS[k])

    plsc.subcore_barrier()
    pltpu.sync_copy(dacc.at[pl.ds(sid * RPT, RPT)], zbuf)
    pltpu.sync_copy(zbuf, out_hbm.at[cid, pl.ds(sid * RPT, RPT)])


def _sc_degree(dst, w, zeros_flat):
    return pl.kernel(
        _deg_kernel,
        out_type=jax.ShapeDtypeStruct((NC, NPAD), jnp.float32),
        mesh=_sc_mesh(),
        compiler_params=pltpu.CompilerParams(needs_layout_passes=False),
        scratch_types=(
            [pltpu.VMEM_SHARED((NPAD,), jnp.float32)]
            + [pltpu.VMEM((CH,), jnp.int32) for _ in range(4)]
            + [pltpu.VMEM((CH,), jnp.float32) for _ in range(4)]
            + [pltpu.VMEM((RPT,), jnp.float32)]
            + [pltpu.SemaphoreType.DMA for _ in range(8)]
        ),
    )(dst, w, zeros_flat)


NCH = 2560
EPAD = NCH * CH
CPW = NCH // NW
NB = CPW


BKE = CH


def _msg_kernel(hw_hbm, src_hbm, dst_hbm, w_hbm, zrows_hbm, out_hbm,
                acc, g0, g1,
                s0, s1, s2, s3, d0, d1, d2, d3, w0, w1, w2, w3, dbuf,
                sl0, sl1, sl2, sl3, sg0, sg1, ss0, ss1):
    cid = lax.axis_index("c")
    sid = lax.axis_index("s")
    wid = cid * NS + sid
    e0 = wid * CPW * CH

    SETS = ((s0, d0, w0, sl0), (s1, d1, w1, sl1),
            (s2, d2, w2, sl2), (s3, d3, w3, sl3))

    for k in range(16):
        pltpu.sync_copy(zrows_hbm, acc.at[pl.ds(sid * RPT + k * 40, 40)])
    plsc.subcore_barrier()

    def loads(b, st):
        sb, db, wb, sem = st
        off = e0 + BKE * b
        pltpu.async_copy(src_hbm.at[pl.ds(off, BKE)], sb, sem)
        pltpu.async_copy(w_hbm.at[pl.ds(off, BKE)], wb, sem)
        pltpu.async_copy(dst_hbm.at[pl.ds(off, BKE)], db, sem)

    def wait_loads(st):
        sb, db, wb, sem = st
        pltpu.make_async_copy(src_hbm.at[pl.ds(0, BKE)], sb, sem).wait()
        pltpu.make_async_copy(w_hbm.at[pl.ds(0, BKE)], wb, sem).wait()
        pltpu.make_async_copy(dst_hbm.at[pl.ds(0, BKE)], db, sem).wait()

    def gath(b, st, gb, sem):
        pltpu.async_copy(hw_hbm.at[st[0]], gb, sem)

    def wait_gath(st, gb, sem):
        pltpu.make_async_copy(hw_hbm.at[st[0]], gb, sem).wait()

    def scat(st, gb, sem):
        pltpu.async_copy(gb, acc.at[st[1]], sem, add=True)

    def wait_scat(st, gb, sem):
        pltpu.make_async_copy(gb, acc.at[st[1]], sem).wait()

    def scale(gb, wb):
        def body(i, carry):
            e = 4 * i
            spl = [plsc.load_gather(wb, [jnp.full((16,), e + u, jnp.int32)])
                   for u in range(4)]
            for u in range(4):
                for f in range(8):
                    gb[e + u, pl.ds(f * 16, 16)] = (
                        gb[e + u, pl.ds(f * 16, 16)] * spl[u])
            return carry

        lax.fori_loop(0, BKE // 4, body, None)

    def half(u, X0, X1, Y0, Y1):
        wait_gath(X0, g0, sg0)
        scale(g0, X0[2])
        scat(X0, g0, ss0)
        wait_gath(X1, g1, sg1)
        scale(g1, X1[2])
        scat(X1, g1, ss1)
        wait_scat(X0, g0, ss0)
        wait_loads(Y0)
        gath(u + 2, Y0, g0, sg0)
        loads(u + 4, X0)
        wait_scat(X1, g1, ss1)
        wait_loads(Y1)
        gath(u + 3, Y1, g1, sg1)
        loads(u + 5, X1)

    for b in range(4):
        loads(b, SETS[b])
    wait_loads(SETS[0])
    gath(0, SETS[0], g0, sg0)
    wait_loads(SETS[1])
    gath(1, SETS[1], g1, sg1)

    def quad(q, _):
        u = 4 * q
        half(u, SETS[0], SETS[1], SETS[2], SETS[3])
        half(u + 2, SETS[2], SETS[3], SETS[0], SETS[1])
        return _

    lax.fori_loop(0, (NB - 4) // 4, quad, None)

    half(NB - 4, SETS[0], SETS[1], SETS[2], SETS[3])

    X0, X1 = SETS[2], SETS[3]
    wait_gath(X0, g0, sg0)
    scale(g0, X0[2])
    scat(X0, g0, ss0)
    wait_gath(X1, g1, sg1)
    scale(g1, X1[2])
    scat(X1, g1, ss1)
    wait_scat(X0, g0, ss0)
    wait_scat(X1, g1, ss1)
    wait_loads(SETS[0])
    wait_loads(SETS[1])

    plsc.subcore_barrier()
    for k in range(16):
        sl = pl.ds(sid * RPT + k * 40, 40)
        pltpu.sync_copy(acc.at[sl], out_hbm.at[cid, sl])


def _sc_message(hw, src2d, dst2d, w2d, zrows):
    return pl.kernel(
        _msg_kernel,
        out_type=jax.ShapeDtypeStruct((NC, NPAD, H), jnp.float32),
        mesh=_sc_mesh(),
        compiler_params=pltpu.CompilerParams(needs_layout_passes=False),
        scratch_types=[
            pltpu.VMEM_SHARED((NPAD, H), jnp.float32),
            pltpu.VMEM((BKE, H), jnp.float32),
            pltpu.VMEM((BKE, H), jnp.float32),
            pltpu.VMEM((BKE,), jnp.int32),
            pltpu.VMEM((BKE,), jnp.int32),
            pltpu.VMEM((BKE,), jnp.int32),
            pltpu.VMEM((BKE,), jnp.int32),
            pltpu.VMEM((BKE,), jnp.int32),
            pltpu.VMEM((BKE,), jnp.int32),
            pltpu.VMEM((BKE,), jnp.int32),
            pltpu.VMEM((BKE,), jnp.int32),
            pltpu.VMEM((BKE,), jnp.float32),
            pltpu.VMEM((BKE,), jnp.float32),
            pltpu.VMEM((BKE,), jnp.float32),
            pltpu.VMEM((BKE,), jnp.float32),
            pltpu.VMEM((40, H), jnp.float32),
            pltpu.SemaphoreType.DMA,
            pltpu.SemaphoreType.DMA,
            pltpu.SemaphoreType.DMA,
            pltpu.SemaphoreType.DMA,
            pltpu.SemaphoreType.DMA,
            pltpu.SemaphoreType.DMA,
            pltpu.SemaphoreType.DMA,
            pltpu.SemaphoreType.DMA,
        ],
    )(hw, src2d, dst2d, w2d, zrows)


def _gru_body(x_ref, wih_ref, whh_ref, bih_ref, bhh_ref, out_ref):
    h = jnp.zeros((R, H), dtype=jnp.float32)
    bih = bih_ref[...]
    bhh = bhh_ref[...]
    wih = wih_ref[...]
    whh = whh_ref[...]
    for t in range(SEQ):
        xt = x_ref[:, t * IN_DIM:(t + 1) * IN_DIM].astype(jnp.bfloat16)
        gi = jnp.dot(xt, wih, preferred_element_type=jnp.float32) + bih
        gh = jnp.dot(h.astype(jnp.bfloat16), whh,
                     preferred_element_type=jnp.float32) + bhh
        r = jax.nn.sigmoid(gi[:, :H] + gh[:, :H])
        z = jax.nn.sigmoid(gi[:, H:2 * H] + gh[:, H:2 * H])
        n = jnp.tanh(gi[:, 2 * H:] + r * gh[:, 2 * H:])
        h = (1.0 - z) * n + z * h
    out_ref[...] = h


def _tc_gru(x, wih_t, whh_t, bih, bhh):
    return pl.pallas_call(
        _gru_body,
        grid=(N // R,),
        in_specs=[
            pl.BlockSpec((R, SEQ * IN_DIM), lambda i: (i, 0)),
            pl.BlockSpec((IN_DIM, 3 * H), lambda i: (0, 0)),
            pl.BlockSpec((H, 3 * H), lambda i: (0, 0)),
            pl.BlockSpec((1, 3 * H), lambda i: (0, 0)),
            pl.BlockSpec((1, 3 * H), lambda i: (0, 0)),
        ],
        out_specs=pl.BlockSpec((R, H), lambda i: (i, 0)),
        out_shape=jax.ShapeDtypeStruct((N, H), jnp.float32),
    )(x, wih_t, whh_t, bih, bhh)


def _prescale_body(ht_ref, p0_ref, p1_ref, w1t_ref, hw_ref, dis_ref):
    deg = p0_ref[...] + p1_ref[...] + 1.0
    dis = lax.rsqrt(deg)
    hw = jnp.dot(ht_ref[...], w1t_ref[...], preferred_element_type=jnp.float32)
    hw_ref[...] = hw * dis
    dis_ref[...] = dis


def _tc_prescale(h_temp, p0, p1, w1t):
    return pl.pallas_call(
        _prescale_body,
        grid=(N // R,),
        in_specs=[
            pl.BlockSpec((R, H), lambda i: (i, 0)),
            pl.BlockSpec((R, 1), lambda i: (i, 0)),
            pl.BlockSpec((R, 1), lambda i: (i, 0)),
            pl.BlockSpec((H, H), lambda i: (0, 0)),
        ],
        out_specs=[
            pl.BlockSpec((R, H), lambda i: (i, 0)),
            pl.BlockSpec((R, 1), lambda i: (i, 0)),
        ],
        out_shape=[
            jax.ShapeDtypeStruct((N, H), jnp.float32),
            jax.ShapeDtypeStruct((N, 1), jnp.float32),
        ],
    )(h_temp, p0, p1, w1t)


def _mid_body(pa_ref, pb_ref, hw1_ref, dis_ref, b1_ref, w2t_ref, out_ref):
    dis = dis_ref[...]
    out1 = jax.nn.relu(dis * (pa_ref[...] + pb_ref[...] + hw1_ref[...]) + b1_ref[...])
    hw2 = jnp.dot(out1, w2t_ref[...], preferred_element_type=jnp.float32)
    out_ref[...] = hw2 * dis


def _tc_mid(pa, pb, hw1p, dis, b1, w2t):
    return pl.pallas_call(
        _mid_body,
        grid=(N // R,),
        in_specs=[
            pl.BlockSpec((R, H), lambda i: (i, 0)),
            pl.BlockSpec((R, H), lambda i: (i, 0)),
            pl.BlockSpec((R, H), lambda i: (i, 0)),
            pl.BlockSpec((R, 1), lambda i: (i, 0)),
            pl.BlockSpec((1, H), lambda i: (0, 0)),
            pl.BlockSpec((H, H), lambda i: (0, 0)),
        ],
        out_specs=pl.BlockSpec((R, H), lambda i: (i, 0)),
        out_shape=jax.ShapeDtypeStruct((N, H), jnp.float32),
    )(pa, pb, hw1p, dis, b1, w2t)


def _head_body(pa_ref, pb_ref, hw2_ref, dis_ref, b2_ref, ht_ref,
               wm1h_ref, wm1o_ref, bm1_ref, wm2_ref, bm2_ref, out_ref):
    dis = dis_ref[...]
    out2 = jax.nn.relu(dis * (pa_ref[...] + pb_ref[...] + hw2_ref[...]) + b2_ref[...])
    hid = jnp.dot(ht_ref[...], wm1h_ref[...], preferred_element_type=jnp.float32)
    hid = hid + jnp.dot(out2, wm1o_ref[...], preferred_element_type=jnp.float32)
    hid = jax.nn.relu(hid + bm1_ref[...])
    out_ref[...] = jnp.dot(hid, wm2_ref[...], preferred_element_type=jnp.float32) + bm2_ref[...]


def _tc_head(pa, pb, hw2p, dis, b2, h_temp, wm1h, wm1o, bm1, wm2t, bm2):
    return pl.pallas_call(
        _head_body,
        grid=(N // R,),
        in_specs=[
            pl.BlockSpec((R, H), lambda i: (i, 0)),
            pl.BlockSpec((R, H), lambda i: (i, 0)),
            pl.BlockSpec((R, H), lambda i: (i, 0)),
            pl.BlockSpec((R, 1), lambda i: (i, 0)),
            pl.BlockSpec((1, H), lambda i: (0, 0)),
            pl.BlockSpec((R, H), lambda i: (i, 0)),
            pl.BlockSpec((H, H), lambda i: (0, 0)),
            pl.BlockSpec((H, H), lambda i: (0, 0)),
            pl.BlockSpec((1, H), lambda i: (0, 0)),
            pl.BlockSpec((H, 1), lambda i: (0, 0)),
            pl.BlockSpec((1, 1), lambda i: (0, 0)),
        ],
        out_specs=pl.BlockSpec((R, 1), lambda i: (i, 0)),
        out_shape=jax.ShapeDtypeStruct((N, 1), jnp.float32),
    )(pa, pb, hw2p, dis, b2, h_temp, wm1h, wm1o, bm1, wm2t, bm2)


def kernel(x, edge_index, edge_weight, W_ih, W_hh, b_ih, b_hh,
           W1, b1, W2, b2, Wm1, bm1, Wm2, bm2):
    src = edge_index[0]
    dst = edge_index[1]
    pad = EPAD + 2 * BKE - E
    pad_idx = jnp.arange(pad, dtype=dst.dtype)
    srcp = jnp.concatenate([src, pad_idx % N])
    dstp = jnp.concatenate([dst, N + pad_idx % (NPAD - N)])
    wp = jnp.concatenate([edge_weight, jnp.zeros((pad,), edge_weight.dtype)])
    zeros_flat = jnp.zeros((RPT,), dtype=jnp.float32)
    zrows = jnp.zeros((40, H), dtype=jnp.float32)

    h_temp = _tc_gru(
        x, W_ih.T.astype(jnp.bfloat16), W_hh.T.astype(jnp.bfloat16),
        b_ih.reshape(1, -1), b_hh.reshape(1, -1)
    )
    degp = _sc_degree(dstp, wp, zeros_flat)
    p0 = degp[0, :N, None]
    p1 = degp[1, :N, None]

    hw1p, dis = _tc_prescale(h_temp, p0, p1, W1.T)

    m1 = _sc_message(hw1p, srcp, dstp, wp, zrows)
    hw2p = _tc_mid(m1[0, :N], m1[1, :N], hw1p, dis, b1.reshape(1, -1), W2.T)

    m2 = _sc_message(hw2p, srcp, dstp, wp, zrows)
    wm1t = Wm1.T
    logits = _tc_head(
        m2[0, :N], m2[1, :N], hw2p, dis, b2.reshape(1, -1), h_temp,
        wm1t[:H], wm1t[H:], bm1.reshape(1, -1), Wm2.T, bm2.reshape(1, 1)
    )
    return logits[:, 0]


# speedup vs baseline: 1.0938x; 1.0938x over previous
import functools

import jax
import jax.numpy as jnp
from jax import lax
from jax.experimental import pallas as pl
from jax.experimental.pallas import tpu as pltpu
from jax.experimental.pallas import tpu_sc as plsc

N = 10000
E = 320000
SEQ = 8
IN_DIM = 16
H = 128

NC = 2
NS = 16
NW = NC * NS
EPW = E // NW
NPAD = 10240
RPT = NPAD // NS
CH = 128
NFULL = EPW // CH
TAIL = EPW - NFULL * CH

R = 1000


def _sc_mesh():
    return plsc.VectorSubcoreMesh(
        core_axis_name="c", subcore_axis_name="s", num_cores=NC, num_subcores=NS
    )


def _deg_kernel(dst_hbm, w_hbm, zeros_hbm, out_hbm, dacc,
                dv0, dv1, dv2, dv3, wv0, wv1, wv2, wv3, zbuf,
                dl0, dl1, dl2, dl3, ds0, ds1, ds2, ds3):
    cid = lax.axis_index("c")
    sid = lax.axis_index("s")
    wid = cid * NS + sid
    e0 = wid * CPW * CH

    SETS = ((dv0, wv0, dl0, ds0), (dv1, wv1, dl1, ds1),
            (dv2, wv2, dl2, ds2), (dv3, wv3, dl3, ds3))

    def loads(b, st):
        dv, wv, sl, _ = st
        off = e0 + CH * b
        pltpu.async_copy(dst_hbm.at[pl.ds(off, CH)], dv, sl)
        pltpu.async_copy(w_hbm.at[pl.ds(off, CH)], wv, sl)

    def wait_loads(st):
        dv, wv, sl, _ = st
        pltpu.make_async_copy(dst_hbm.at[pl.ds(0, CH)], dv, sl).wait()
        pltpu.make_async_copy(w_hbm.at[pl.ds(0, CH)], wv, sl).wait()

    def scat(st):
        dv, wv, _, ss = st
        pltpu.async_copy(wv, dacc.at[dv], ss, add=True)

    def wait_scat(st):
        dv, wv, _, ss = st
        pltpu.make_async_copy(wv, dacc.at[dv], ss).wait()

    for k in range(4):
        loads(k, SETS[k])
    pltpu.sync_copy(zeros_hbm.at[pl.ds(0, RPT)], zbuf)
    pltpu.sync_copy(zbuf, dacc.at[pl.ds(sid * RPT, RPT)])
    plsc.subcore_barrier()

    def quad(q, _):
        b = 4 * q
        for k in range(4):
            wait_loads(SETS[k])
            scat(SETS[k])
        for k in range(4):
            wait_scat(SETS[k])
            loads(b + 4 + k, SETS[k])
        return _

    lax.fori_loop(0, CPW // 4 - 1, quad, None)
    for k in range(4):
        wait_loads(SETS[k])
        scat(SETS[k])
    for k in range(4):
        wait_scat(SETS[k])

    plsc.subcore_barrier()
    pltpu.sync_copy(dacc.at[pl.ds(sid * RPT, RPT)], zbuf)
    pltpu.sync_copy(zbuf, out_hbm.at[cid, pl.ds(sid * RPT, RPT)])


def _sc_degree(dst, w, zeros_flat):
    return pl.kernel(
        _deg_kernel,
        out_type=jax.ShapeDtypeStruct((NC, NPAD), jnp.float32),
        mesh=_sc_mesh(),
        compiler_params=pltpu.CompilerParams(needs_layout_passes=False),
        scratch_types=(
            [pltpu.VMEM_SHARED((NPAD,), jnp.float32)]
            + [pltpu.VMEM((CH,), jnp.int32) for _ in range(4)]
            + [pltpu.VMEM((CH,), jnp.float32) for _ in range(4)]
            + [pltpu.VMEM((RPT,), jnp.float32)]
            + [pltpu.SemaphoreType.DMA for _ in range(8)]
        ),
    )(dst, w, zeros_flat)


NCH = 2560
EPAD = NCH * CH
CPW = NCH // NW
NB = CPW


BKE = CH


def _msg_kernel(hw_hbm, src_hbm, dst_hbm, w_hbm, zrows_hbm, out_hbm,
                acc, g0, g1,
                s0, s1, s2, s3, d0, d1, d2, d3, w0, w1, w2, w3, dbuf,
                sl0, sl1, sl2, sl3, sg0, sg1, ss0, ss1):
    cid = lax.axis_index("c")
    sid = lax.axis_index("s")
    wid = cid * NS + sid
    e0 = wid * CPW * CH

    SETS = ((s0, d0, w0, sl0), (s1, d1, w1, sl1),
            (s2, d2, w2, sl2), (s3, d3, w3, sl3))

    pltpu.sync_copy(zrows_hbm, dbuf)
    for k in range(16):
        pltpu.sync_copy(dbuf, acc.at[pl.ds(sid * RPT + k * 40, 40)])
    plsc.subcore_barrier()

    def loads(b, st):
        sb, db, wb, sem = st
        off = e0 + BKE * b
        pltpu.async_copy(src_hbm.at[pl.ds(off, BKE)], sb, sem)
        pltpu.async_copy(w_hbm.at[pl.ds(off, BKE)], wb, sem)
        pltpu.async_copy(dst_hbm.at[pl.ds(off, BKE)], db, sem)

    def wait_loads(st):
        sb, db, wb, sem = st
        pltpu.make_async_copy(src_hbm.at[pl.ds(0, BKE)], sb, sem).wait()
        pltpu.make_async_copy(w_hbm.at[pl.ds(0, BKE)], wb, sem).wait()
        pltpu.make_async_copy(dst_hbm.at[pl.ds(0, BKE)], db, sem).wait()

    def gath(b, st, gb, sem):
        pltpu.async_copy(hw_hbm.at[st[0]], gb, sem)

    def wait_gath(st, gb, sem):
        pltpu.make_async_copy(hw_hbm.at[st[0]], gb, sem).wait()

    def scat(st, gb, sem):
        pltpu.async_copy(gb, acc.at[st[1]], sem, add=True)

    def wait_scat(st, gb, sem):
        pltpu.make_async_copy(gb, acc.at[st[1]], sem).wait()

    def scale(gb, wb):
        def body(i, carry):
            e = 2 * i
            spl0 = plsc.load_gather(wb, [jnp.full((16,), e, jnp.int32)])
            spl1 = plsc.load_gather(wb, [jnp.full((16,), e + 1, jnp.int32)])
            for f in range(8):
                gb[e, pl.ds(f * 16, 16)] = gb[e, pl.ds(f * 16, 16)] * spl0
            for f in range(8):
                gb[e + 1, pl.ds(f * 16, 16)] = gb[e + 1, pl.ds(f * 16, 16)] * spl1
            return carry

        lax.fori_loop(0, BKE // 2, body, None)

    def half(u, X0, X1, Y0, Y1):
        wait_gath(X0, g0, sg0)
        scale(g0, X0[2])
        scat(X0, g0, ss0)
        wait_gath(X1, g1, sg1)
        scale(g1, X1[2])
        scat(X1, g1, ss1)
        wait_scat(X0, g0, ss0)
        wait_loads(Y0)
        gath(u + 2, Y0, g0, sg0)
        loads(u + 4, X0)
        wait_scat(X1, g1, ss1)
        wait_loads(Y1)
        gath(u + 3, Y1, g1, sg1)
        loads(u + 5, X1)

    for b in range(4):
        loads(b, SETS[b])
    wait_loads(SETS[0])
    gath(0, SETS[0], g0, sg0)
    wait_loads(SETS[1])
    gath(1, SETS[1], g1, sg1)

    def quad(q, _):
        u = 4 * q
        half(u, SETS[0], SETS[1], SETS[2], SETS[3])
        half(u + 2, SETS[2], SETS[3], SETS[0], SETS[1])
        return _

    lax.fori_loop(0, (NB - 4) // 4, quad, None)

    half(NB - 4, SETS[0], SETS[1], SETS[2], SETS[3])

    X0, X1 = SETS[2], SETS[3]
    wait_gath(X0, g0, sg0)
    scale(g0, X0[2])
    scat(X0, g0, ss0)
    wait_gath(X1, g1, sg1)
    scale(g1, X1[2])
    scat(X1, g1, ss1)
    wait_scat(X0, g0, ss0)
    wait_scat(X1, g1, ss1)
    wait_loads(SETS[0])
    wait_loads(SETS[1])

    plsc.subcore_barrier()
    for k in range(16):
        sl = pl.ds(sid * RPT + k * 40, 40)
        pltpu.sync_copy(acc.at[sl], dbuf)
        pltpu.sync_copy(dbuf, out_hbm.at[cid, sl])


def _sc_message(hw, src2d, dst2d, w2d, zrows):
    return pl.kernel(
        _msg_kernel,
        out_type=jax.ShapeDtypeStruct((NC, NPAD, H), jnp.float32),
        mesh=_sc_mesh(),
        compiler_params=pltpu.CompilerParams(needs_layout_passes=False),
        scratch_types=[
            pltpu.VMEM_SHARED((NPAD, H), jnp.float32),
            pltpu.VMEM((BKE, H), jnp.float32),
            pltpu.VMEM((BKE, H), jnp.float32),
            pltpu.VMEM((BKE,), jnp.int32),
            pltpu.VMEM((BKE,), jnp.int32),
            pltpu.VMEM((BKE,), jnp.int32),
            pltpu.VMEM((BKE,), jnp.int32),
            pltpu.VMEM((BKE,), jnp.int32),
            pltpu.VMEM((BKE,), jnp.int32),
            pltpu.VMEM((BKE,), jnp.int32),
            pltpu.VMEM((BKE,), jnp.int32),
            pltpu.VMEM((BKE,), jnp.float32),
            pltpu.VMEM((BKE,), jnp.float32),
            pltpu.VMEM((BKE,), jnp.float32),
            pltpu.VMEM((BKE,), jnp.float32),
            pltpu.VMEM((40, H), jnp.float32),
            pltpu.SemaphoreType.DMA,
            pltpu.SemaphoreType.DMA,
            pltpu.SemaphoreType.DMA,
            pltpu.SemaphoreType.DMA,
            pltpu.SemaphoreType.DMA,
            pltpu.SemaphoreType.DMA,
            pltpu.SemaphoreType.DMA,
            pltpu.SemaphoreType.DMA,
        ],
    )(hw, src2d, dst2d, w2d, zrows)


def _gru_body(x_ref, wih_ref, whh_ref, bih_ref, bhh_ref, out_ref):
    h = jnp.zeros((R, H), dtype=jnp.float32)
    bih = bih_ref[...]
    bhh = bhh_ref[...]
    wih = wih_ref[...]
    whh = whh_ref[...]
    for t in range(SEQ):
        xt = x_ref[:, t * IN_DIM:(t + 1) * IN_DIM].astype(jnp.bfloat16)
        gi = jnp.dot(xt, wih, preferred_element_type=jnp.float32) + bih
        gh = jnp.dot(h.astype(jnp.bfloat16), whh,
                     preferred_element_type=jnp.float32) + bhh
        r = jax.nn.sigmoid(gi[:, :H] + gh[:, :H])
        z = jax.nn.sigmoid(gi[:, H:2 * H] + gh[:, H:2 * H])
        n = jnp.tanh(gi[:, 2 * H:] + r * gh[:, 2 * H:])
        h = (1.0 - z) * n + z * h
    out_ref[...] = h


def _tc_gru(x, wih_t, whh_t, bih, bhh):
    return pl.pallas_call(
        _gru_body,
        grid=(N // R,),
        in_specs=[
            pl.BlockSpec((R, SEQ * IN_DIM), lambda i: (i, 0)),
            pl.BlockSpec((IN_DIM, 3 * H), lambda i: (0, 0)),
            pl.BlockSpec((H, 3 * H), lambda i: (0, 0)),
            pl.BlockSpec((1, 3 * H), lambda i: (0, 0)),
            pl.BlockSpec((1, 3 * H), lambda i: (0, 0)),
        ],
        out_specs=pl.BlockSpec((R, H), lambda i: (i, 0)),
        out_shape=jax.ShapeDtypeStruct((N, H), jnp.float32),
    )(x, wih_t, whh_t, bih, bhh)


def _prescale_body(ht_ref, p0_ref, p1_ref, w1t_ref, hw_ref, dis_ref):
    deg = p0_ref[...] + p1_ref[...] + 1.0
    dis = lax.rsqrt(deg)
    hw = jnp.dot(ht_ref[...], w1t_ref[...], preferred_element_type=jnp.float32)
    hw_ref[...] = hw * dis
    dis_ref[...] = dis


def _tc_prescale(h_temp, p0, p1, w1t):
    return pl.pallas_call(
        _prescale_body,
        grid=(N // R,),
        in_specs=[
            pl.BlockSpec((R, H), lambda i: (i, 0)),
            pl.BlockSpec((R, 1), lambda i: (i, 0)),
            pl.BlockSpec((R, 1), lambda i: (i, 0)),
            pl.BlockSpec((H, H), lambda i: (0, 0)),
        ],
        out_specs=[
            pl.BlockSpec((R, H), lambda i: (i, 0)),
            pl.BlockSpec((R, 1), lambda i: (i, 0)),
        ],
        out_shape=[
            jax.ShapeDtypeStruct((N, H), jnp.float32),
            jax.ShapeDtypeStruct((N, 1), jnp.float32),
        ],
    )(h_temp, p0, p1, w1t)


def _mid_body(pa_ref, pb_ref, hw1_ref, dis_ref, b1_ref, w2t_ref, out_ref):
    dis = dis_ref[...]
    out1 = jax.nn.relu(dis * (pa_ref[...] + pb_ref[...] + hw1_ref[...]) + b1_ref[...])
    hw2 = jnp.dot(out1, w2t_ref[...], preferred_element_type=jnp.float32)
    out_ref[...] = hw2 * dis


def _tc_mid(pa, pb, hw1p, dis, b1, w2t):
    return pl.pallas_call(
        _mid_body,
        grid=(N // R,),
        in_specs=[
            pl.BlockSpec((R, H), lambda i: (i, 0)),
            pl.BlockSpec((R, H), lambda i: (i, 0)),
            pl.BlockSpec((R, H), lambda i: (i, 0)),
            pl.BlockSpec((R, 1), lambda i: (i, 0)),
            pl.BlockSpec((1, H), lambda i: (0, 0)),
            pl.BlockSpec((H, H), lambda i: (0, 0)),
        ],
        out_specs=pl.BlockSpec((R, H), lambda i: (i, 0)),
        out_shape=jax.ShapeDtypeStruct((N, H), jnp.float32),
    )(pa, pb, hw1p, dis, b1, w2t)


def _head_body(pa_ref, pb_ref, hw2_ref, dis_ref, b2_ref, ht_ref,
               wm1h_ref, wm1o_ref, bm1_ref, wm2_ref, bm2_ref, out_ref):
    dis = dis_ref[...]
    out2 = jax.nn.relu(dis * (pa_ref[...] + pb_ref[...] + hw2_ref[...]) + b2_ref[...])
    hid = jnp.dot(ht_ref[...], wm1h_ref[...], preferred_element_type=jnp.float32)
    hid = hid + jnp.dot(out2, wm1o_ref[...], preferred_element_type=jnp.float32)
    hid = jax.nn.relu(hid + bm1_ref[...])
    out_ref[...] = jnp.dot(hid, wm2_ref[...], preferred_element_type=jnp.float32) + bm2_ref[...]


def _tc_head(pa, pb, hw2p, dis, b2, h_temp, wm1h, wm1o, bm1, wm2t, bm2):
    return pl.pallas_call(
        _head_body,
        grid=(N // R,),
        in_specs=[
            pl.BlockSpec((R, H), lambda i: (i, 0)),
            pl.BlockSpec((R, H), lambda i: (i, 0)),
            pl.BlockSpec((R, H), lambda i: (i, 0)),
            pl.BlockSpec((R, 1), lambda i: (i, 0)),
            pl.BlockSpec((1, H), lambda i: (0, 0)),
            pl.BlockSpec((R, H), lambda i: (i, 0)),
            pl.BlockSpec((H, H), lambda i: (0, 0)),
            pl.BlockSpec((H, H), lambda i: (0, 0)),
            pl.BlockSpec((1, H), lambda i: (0, 0)),
            pl.BlockSpec((H, 1), lambda i: (0, 0)),
            pl.BlockSpec((1, 1), lambda i: (0, 0)),
        ],
        out_specs=pl.BlockSpec((R, 1), lambda i: (i, 0)),
        out_shape=jax.ShapeDtypeStruct((N, 1), jnp.float32),
    )(pa, pb, hw2p, dis, b2, h_temp, wm1h, wm1o, bm1, wm2t, bm2)


def kernel(x, edge_index, edge_weight, W_ih, W_hh, b_ih, b_hh,
           W1, b1, W2, b2, Wm1, bm1, Wm2, bm2):
    src = edge_index[0]
    dst = edge_index[1]
    pad = EPAD + 2 * BKE - E
    pad_idx = jnp.arange(pad, dtype=dst.dtype)
    srcp = jnp.concatenate([src, pad_idx % N])
    dstp = jnp.concatenate([dst, N + pad_idx % (NPAD - N)])
    wp = jnp.concatenate([edge_weight, jnp.zeros((pad,), edge_weight.dtype)])
    zeros_flat = jnp.zeros((RPT,), dtype=jnp.float32)
    zrows = jnp.zeros((40, H), dtype=jnp.float32)

    h_temp = _tc_gru(
        x, W_ih.T.astype(jnp.bfloat16), W_hh.T.astype(jnp.bfloat16),
        b_ih.reshape(1, -1), b_hh.reshape(1, -1)
    )
    degp = _sc_degree(dstp, wp, zeros_flat)
    p0 = degp[0, :N, None]
    p1 = degp[1, :N, None]

    hw1p, dis = _tc_prescale(h_temp, p0, p1, W1.T)

    m1 = _sc_message(hw1p, srcp, dstp, wp, zrows)
    hw2p = _tc_mid(m1[0, :N], m1[1, :N], hw1p, dis, b1.reshape(1, -1), W2.T)

    m2 = _sc_message(hw2p, srcp, dstp, wp, zrows)
    wm1t = Wm1.T
    logits = _tc_head(
        m2[0, :N], m2[1, :N], hw2p, dis, b2.reshape(1, -1), h_temp,
        wm1t[:H], wm1t[H:], bm1.reshape(1, -1), Wm2.T, bm2.reshape(1, 1)
    )
    return logits[:, 0]
